# baseline (device time: 36852 ns/iter reference)
import jax
import jax.numpy as jnp
from jax import lax
from jax.experimental import pallas as pl
from jax.experimental.pallas import tpu as pltpu

N_DEV = 32
N_PLANE = 8
N_Z = 4
N_HALF = 4


def kernel(A, B):
    m, k = A.shape
    _, n = B.shape
    chunk = m // N_DEV
    grp = N_Z * chunk
    nh = n // N_HALF

    def body(a_ref, b_ref, out_ref, ab_ref, bb_ref,
             sendA_ref, commA_ref, sendB_ref, commB_ref,
             sendA_sems, recvA_sems, sendB_sems, recvB_sems):
        my_id = lax.axis_index("i")
        my_z = my_id // N_PLANE
        my_p = my_id % N_PLANE

        plane_peers = [my_z * N_PLANE + (my_p + s) % N_PLANE
                       for s in range(1, N_PLANE)]
        z_peers = [((my_z + s) % N_Z) * N_PLANE + my_p
                   for s in range(1, N_Z)]
        n_peers = len(plane_peers) + len(z_peers)

        barrier_sem = pltpu.get_barrier_semaphore()
        for peer in plane_peers + z_peers:
            pl.semaphore_signal(
                barrier_sem, inc=1,
                device_id=(peer,), device_id_type=pl.DeviceIdType.MESH,
            )

        bb_ref[...] = b_ref[...].astype(jnp.bfloat16)
        for q in range(N_PLANE):
            j = (my_p + 1 + q) % N_PLANE
            for zo in range(N_Z):
                ab_ref[q * grp + zo * chunk:q * grp + (zo + 1) * chunk, :] = (
                    a_ref[pl.ds(zo * (m // N_Z) + j * chunk, chunk), :]
                    .astype(jnp.bfloat16)
                )

        rdmas_a = []
        own_f32 = [None] * N_HALF
        for h in range(N_HALF):
            for r in range(N_Z):
                blk = jnp.dot(
                    ab_ref[r * 2 * grp:(r + 1) * 2 * grp, :],
                    bb_ref[:, h * nh:(h + 1) * nh],
                    preferred_element_type=jnp.float32,
                )
                for half_blk in range(2):
                    q = 2 * r + half_blk
                    j = (my_p + 1 + q) % N_PLANE
                    part = blk[half_blk * grp:(half_blk + 1) * grp]
                    if q == N_PLANE - 1:
                        own_f32[h] = part.reshape(N_Z, chunk, nh)
                        continue
                    slot = h * N_PLANE + j
                    sendA_ref[pl.ds(slot, 1)] = (
                        part.astype(jnp.bfloat16).reshape(1, N_Z, chunk, nh)
                    )
                    if not rdmas_a:
                        pl.semaphore_wait(barrier_sem, n_peers)
                    rdma = pltpu.make_async_remote_copy(
                        src_ref=sendA_ref.at[pl.ds(slot, 1)],
                        dst_ref=commA_ref.at[pl.ds(h * N_PLANE + my_p, 1)],
                        send_sem=sendA_sems.at[slot],
                        recv_sem=recvA_sems.at[h * N_PLANE + my_p],
                        device_id=(my_z * N_PLANE + j,),
                        device_id_type=pl.DeviceIdType.MESH,
                    )
                    rdma.start()
                    rdmas_a.append(rdma)

        rdmas_b = []
        for h in range(N_HALF):
            accA = own_f32[h]
            for s in range(1, N_PLANE):
                j = (my_p - s) % N_PLANE
                slot = h * N_PLANE + j
                recv = pltpu.make_async_remote_copy(
                    src_ref=sendA_ref.at[pl.ds(slot, 1)],
                    dst_ref=commA_ref.at[pl.ds(slot, 1)],
                    send_sem=sendA_sems.at[slot],
                    recv_sem=recvA_sems.at[slot],
                    device_id=(0,),
                    device_id_type=pl.DeviceIdType.MESH,
                )
                recv.wait_recv()
                accA = accA + commA_ref[pl.ds(slot, 1)].astype(
                    jnp.float32).reshape(N_Z, chunk, nh)
            sendB_ref[h * N_Z:(h + 1) * N_Z] = accA

            for s in range(1, N_Z):
                zt = (my_z + s) % N_Z
                slot = h * N_Z + zt
                rdma = pltpu.make_async_remote_copy(
                    src_ref=sendB_ref.at[pl.ds(slot, 1)],
                    dst_ref=commB_ref.at[pl.ds(h * N_Z + my_z, 1)],
                    send_sem=sendB_sems.at[slot],
                    recv_sem=recvB_sems.at[h * N_Z + my_z],
                    device_id=(zt * N_PLANE + my_p,),
                    device_id_type=pl.DeviceIdType.MESH,
                )
                rdma.start()
                rdmas_b.append(rdma)

        for h in range(N_HALF):
            acc = sendB_ref[pl.ds(h * N_Z + my_z, 1)].reshape(chunk, nh)
            for s in range(1, N_Z):
                zt = (my_z - s) % N_Z
                slot = h * N_Z + zt
                recv = pltpu.make_async_remote_copy(
                    src_ref=sendB_ref.at[pl.ds(slot, 1)],
                    dst_ref=commB_ref.at[pl.ds(slot, 1)],
                    send_sem=sendB_sems.at[slot],
                    recv_sem=recvB_sems.at[slot],
                    device_id=(0,),
                    device_id_type=pl.DeviceIdType.MESH,
                )
                recv.wait_recv()
                acc = acc + commB_ref[pl.ds(slot, 1)].reshape(chunk, nh)
            out_ref[:, h * nh:(h + 1) * nh] = acc

        for rdma in rdmas_a + rdmas_b:
            rdma.wait_send()

    return pl.pallas_call(
        body,
        out_shape=jax.ShapeDtypeStruct((chunk, n), jnp.float32),
        in_specs=[
            pl.BlockSpec(memory_space=pltpu.VMEM),
            pl.BlockSpec(memory_space=pltpu.VMEM),
        ],
        out_specs=pl.BlockSpec(memory_space=pltpu.VMEM),
        scratch_shapes=[
            pltpu.VMEM((m, k), jnp.bfloat16),
            pltpu.VMEM((k, n), jnp.bfloat16),
            pltpu.VMEM((N_HALF * N_PLANE, N_Z, chunk, nh),
                       jnp.bfloat16),
            pltpu.VMEM((N_HALF * N_PLANE, N_Z, chunk, nh),
                       jnp.bfloat16),
            pltpu.VMEM((N_HALF * N_Z, chunk, nh), jnp.float32),
            pltpu.VMEM((N_HALF * N_Z, chunk, nh), jnp.float32),
            pltpu.SemaphoreType.DMA((N_HALF * N_PLANE,)),
            pltpu.SemaphoreType.DMA((N_HALF * N_PLANE,)),
            pltpu.SemaphoreType.DMA((N_HALF * N_Z,)),
            pltpu.SemaphoreType.DMA((N_HALF * N_Z,)),
        ],
        compiler_params=pltpu.CompilerParams(collective_id=0),
    )(A, B)


# device time: 36673 ns/iter; 1.0049x vs baseline; 1.0049x over previous
import jax
import jax.numpy as jnp
from jax import lax
from jax.experimental import pallas as pl
from jax.experimental.pallas import tpu as pltpu

N_DEV = 32
N_PLANE = 8
N_Z = 4
N_HALF = 4


def kernel(A, B):
    m, k = A.shape
    _, n = B.shape
    chunk = m // N_DEV
    grp = N_Z * chunk
    nh = n // N_HALF

    def body(a_ref, b_ref, out_ref, ab_ref, bb_ref,
             sendA_ref, commA_ref, sendB_ref, commB_ref,
             sendA_sems, recvA_sems, sendB_sems, recvB_sems):
        my_id = lax.axis_index("i")
        my_z = my_id // N_PLANE
        my_p = my_id % N_PLANE

        plane_peers = [my_z * N_PLANE + (my_p + s) % N_PLANE
                       for s in range(1, N_PLANE)]
        z_peers = [((my_z + s) % N_Z) * N_PLANE + my_p
                   for s in range(1, N_Z)]
        n_peers = len(plane_peers) + len(z_peers)

        barrier_sem = pltpu.get_barrier_semaphore()
        for peer in plane_peers + z_peers:
            pl.semaphore_signal(
                barrier_sem, inc=1,
                device_id=(peer,), device_id_type=pl.DeviceIdType.MESH,
            )

        bb_ref[...] = b_ref[...].astype(jnp.bfloat16)
        for q in range(N_PLANE):
            j = (my_p + 1 + q) % N_PLANE
            for zo in range(N_Z):
                ab_ref[q * grp + zo * chunk:q * grp + (zo + 1) * chunk, :] = (
                    a_ref[pl.ds(zo * (m // N_Z) + j * chunk, chunk), :]
                    .astype(jnp.bfloat16)
                )

        rdmas_a = []
        own_f32 = [None] * N_HALF
        for h in range(N_HALF):
            for r in range(N_Z):
                blk = jnp.dot(
                    ab_ref[r * 2 * grp:(r + 1) * 2 * grp, :],
                    bb_ref[:, h * nh:(h + 1) * nh],
                    preferred_element_type=jnp.float32,
                )
                for half_blk in range(2):
                    q = 2 * r + half_blk
                    j = (my_p + 1 + q) % N_PLANE
                    part = blk[half_blk * grp:(half_blk + 1) * grp]
                    if q == N_PLANE - 1:
                        own_f32[h] = part.reshape(N_Z, chunk, nh)
                        continue
                    slot = h * N_PLANE + j
                    sendA_ref[pl.ds(slot, 1)] = (
                        part.astype(jnp.bfloat16).reshape(1, N_Z, chunk, nh)
                    )
                    if not rdmas_a:
                        pl.semaphore_wait(barrier_sem, n_peers)
                    rdma = pltpu.make_async_remote_copy(
                        src_ref=sendA_ref.at[pl.ds(slot, 1)],
                        dst_ref=commA_ref.at[pl.ds(h * N_PLANE + my_p, 1)],
                        send_sem=sendA_sems.at[slot],
                        recv_sem=recvA_sems.at[h * N_PLANE + my_p],
                        device_id=(my_z * N_PLANE + j,),
                        device_id_type=pl.DeviceIdType.MESH,
                    )
                    rdma.start()
                    rdmas_a.append(rdma)

        rdmas_b = []
        for h in range(N_HALF):
            accA = own_f32[h]
            for s in range(1, N_PLANE):
                j = (my_p - s) % N_PLANE
                slot = h * N_PLANE + j
                recv = pltpu.make_async_remote_copy(
                    src_ref=sendA_ref.at[pl.ds(slot, 1)],
                    dst_ref=commA_ref.at[pl.ds(slot, 1)],
                    send_sem=sendA_sems.at[slot],
                    recv_sem=recvA_sems.at[slot],
                    device_id=(0,),
                    device_id_type=pl.DeviceIdType.MESH,
                )
                recv.wait_recv()
                accA = accA + commA_ref[pl.ds(slot, 1)].astype(
                    jnp.float32).reshape(N_Z, chunk, nh)
            sendB_ref[h * N_Z:(h + 1) * N_Z] = accA.astype(jnp.bfloat16)

            for s in range(1, N_Z):
                zt = (my_z + s) % N_Z
                slot = h * N_Z + zt
                rdma = pltpu.make_async_remote_copy(
                    src_ref=sendB_ref.at[pl.ds(slot, 1)],
                    dst_ref=commB_ref.at[pl.ds(h * N_Z + my_z, 1)],
                    send_sem=sendB_sems.at[slot],
                    recv_sem=recvB_sems.at[h * N_Z + my_z],
                    device_id=(zt * N_PLANE + my_p,),
                    device_id_type=pl.DeviceIdType.MESH,
                )
                rdma.start()
                rdmas_b.append(rdma)

        for h in range(N_HALF):
            acc = sendB_ref[pl.ds(h * N_Z + my_z, 1)].astype(
                jnp.float32).reshape(chunk, nh)
            for s in range(1, N_Z):
                zt = (my_z - s) % N_Z
                slot = h * N_Z + zt
                recv = pltpu.make_async_remote_copy(
                    src_ref=sendB_ref.at[pl.ds(slot, 1)],
                    dst_ref=commB_ref.at[pl.ds(slot, 1)],
                    send_sem=sendB_sems.at[slot],
                    recv_sem=recvB_sems.at[slot],
                    device_id=(0,),
                    device_id_type=pl.DeviceIdType.MESH,
                )
                recv.wait_recv()
                acc = acc + commB_ref[pl.ds(slot, 1)].astype(
                    jnp.float32).reshape(chunk, nh)
            out_ref[:, h * nh:(h + 1) * nh] = acc

        for rdma in rdmas_a + rdmas_b:
            rdma.wait_send()

    return pl.pallas_call(
        body,
        out_shape=jax.ShapeDtypeStruct((chunk, n), jnp.float32),
        in_specs=[
            pl.BlockSpec(memory_space=pltpu.VMEM),
            pl.BlockSpec(memory_space=pltpu.VMEM),
        ],
        out_specs=pl.BlockSpec(memory_space=pltpu.VMEM),
        scratch_shapes=[
            pltpu.VMEM((m, k), jnp.bfloat16),
            pltpu.VMEM((k, n), jnp.bfloat16),
            pltpu.VMEM((N_HALF * N_PLANE, N_Z, chunk, nh),
                       jnp.bfloat16),
            pltpu.VMEM((N_HALF * N_PLANE, N_Z, chunk, nh),
                       jnp.bfloat16),
            pltpu.VMEM((N_HALF * N_Z, chunk, nh), jnp.bfloat16),
            pltpu.VMEM((N_HALF * N_Z, chunk, nh), jnp.bfloat16),
            pltpu.SemaphoreType.DMA((N_HALF * N_PLANE,)),
            pltpu.SemaphoreType.DMA((N_HALF * N_PLANE,)),
            pltpu.SemaphoreType.DMA((N_HALF * N_Z,)),
            pltpu.SemaphoreType.DMA((N_HALF * N_Z,)),
        ],
        compiler_params=pltpu.CompilerParams(collective_id=0),
    )(A, B)


# device time: 30682 ns/iter; 1.2011x vs baseline; 1.1953x over previous
import jax
import jax.numpy as jnp
from jax import lax
from jax.experimental import pallas as pl
from jax.experimental.pallas import tpu as pltpu

N_DEV = 32
N_PLANE = 8
N_Z = 4
N_Y = 4
N_Q = 4


def kernel(A, B):
    m, k = A.shape
    _, n = B.shape
    chunk = m // N_DEV
    nh = n // N_Q
    half_rows = m // 2

    def body(a_ref, b_ref, out_ref, ab_ref, bb_ref,
             sendA1_ref, commA1_ref, ownA_ref,
             sendA2_ref, commA2_ref, sendB_ref, commB_ref,
             a1s_sems, a1r_sems, a2s_sems, a2r_sems, bs_sems, br_sems):
        my_id = lax.axis_index("i")
        my_z = my_id // N_PLANE
        my_p = my_id % N_PLANE
        my_y = my_p // 2
        my_x = (my_p & 1) ^ (my_y & 1)
        partner_p = my_p ^ 1

        def p_of(x, y):
            return 2 * y + (x ^ (y & 1))

        rail_peers = [my_z * N_PLANE + p_of(my_x, (my_y + s) % N_Y)
                      for s in range(1, N_Y)]
        z_peers = [((my_z + s) % N_Z) * N_PLANE + my_p
                   for s in range(1, N_Z)]
        peers = [my_z * N_PLANE + partner_p] + rail_peers + z_peers

        barrier_sem = pltpu.get_barrier_semaphore()
        for peer in peers:
            pl.semaphore_signal(
                barrier_sem, inc=1,
                device_id=(peer,), device_id_type=pl.DeviceIdType.MESH,
            )

        bb_ref[...] = b_ref[...].astype(jnp.bfloat16)
        for side in range(2):
            for y in range(N_Y):
                for zo in range(N_Z):
                    po = p_of((1 - my_x) if side == 0 else my_x, y)
                    dst = side * half_rows + (y * N_Z + zo) * chunk
                    ab_ref[dst:dst + chunk, :] = (
                        a_ref[pl.ds(zo * (m // N_Z) + po * chunk, chunk), :]
                        .astype(jnp.bfloat16)
                    )

        rdmas = []
        for q in range(N_Q):
            blk_opp = jnp.dot(
                ab_ref[:half_rows, :], bb_ref[:, q * nh:(q + 1) * nh],
                preferred_element_type=jnp.float32,
            )
            sendA1_ref[q] = blk_opp.astype(jnp.bfloat16).reshape(
                N_Y, N_Z, chunk, nh)
            if q == 0:
                pl.semaphore_wait(barrier_sem, len(peers))
            a1 = pltpu.make_async_remote_copy(
                src_ref=sendA1_ref.at[q],
                dst_ref=commA1_ref.at[q],
                send_sem=a1s_sems.at[q],
                recv_sem=a1r_sems.at[q],
                device_id=(my_z * N_PLANE + partner_p,),
                device_id_type=pl.DeviceIdType.MESH,
            )
            a1.start()
            rdmas.append(a1)
            blk_own = jnp.dot(
                ab_ref[half_rows:, :], bb_ref[:, q * nh:(q + 1) * nh],
                preferred_element_type=jnp.float32,
            )
            ownA_ref[q] = blk_own.astype(jnp.bfloat16).reshape(
                N_Y, N_Z, chunk, nh)

        for q in range(N_Q):
            a1recv = pltpu.make_async_remote_copy(
                src_ref=sendA1_ref.at[q],
                dst_ref=commA1_ref.at[q],
                send_sem=a1s_sems.at[q],
                recv_sem=a1r_sems.at[q],
                device_id=(0,),
                device_id_type=pl.DeviceIdType.MESH,
            )
            a1recv.wait_recv()
            pair = (commA1_ref[q].astype(jnp.float32)
                    + ownA_ref[q].astype(jnp.float32))
            sendA2_ref[q * N_Y:(q + 1) * N_Y] = pair.astype(jnp.bfloat16)
            for s in range(1, N_Y):
                yt = (my_y + s) % N_Y
                a2 = pltpu.make_async_remote_copy(
                    src_ref=sendA2_ref.at[pl.ds(q * N_Y + yt, 1)],
                    dst_ref=commA2_ref.at[pl.ds(q * N_Y + my_y, 1)],
                    send_sem=a2s_sems.at[q * N_Y + yt],
                    recv_sem=a2r_sems.at[q * N_Y + my_y],
                    device_id=(my_z * N_PLANE + p_of(my_x, yt),),
                    device_id_type=pl.DeviceIdType.MESH,
                )
                a2.start()
                rdmas.append(a2)

        for q in range(N_Q):
            acc = sendA2_ref[pl.ds(q * N_Y + my_y, 1)].astype(
                jnp.float32).reshape(N_Z, chunk, nh)
            for s in range(1, N_Y):
                ys = (my_y - s) % N_Y
                slot = q * N_Y + ys
                a2recv = pltpu.make_async_remote_copy(
                    src_ref=sendA2_ref.at[pl.ds(slot, 1)],
                    dst_ref=commA2_ref.at[pl.ds(slot, 1)],
                    send_sem=a2s_sems.at[slot],
                    recv_sem=a2r_sems.at[slot],
                    device_id=(0,),
                    device_id_type=pl.DeviceIdType.MESH,
                )
                a2recv.wait_recv()
                acc = acc + commA2_ref[pl.ds(slot, 1)].astype(
                    jnp.float32).reshape(N_Z, chunk, nh)
            sendB_ref[q * N_Z:(q + 1) * N_Z] = acc.astype(jnp.bfloat16)
            for s in range(1, N_Z):
                zt = (my_z + s) % N_Z
                b_rdma = pltpu.make_async_remote_copy(
                    src_ref=sendB_ref.at[pl.ds(q * N_Z + zt, 1)],
                    dst_ref=commB_ref.at[pl.ds(q * N_Z + my_z, 1)],
                    send_sem=bs_sems.at[q * N_Z + zt],
                    recv_sem=br_sems.at[q * N_Z + my_z],
                    device_id=(zt * N_PLANE + my_p,),
                    device_id_type=pl.DeviceIdType.MESH,
                )
                b_rdma.start()
                rdmas.append(b_rdma)

        for q in range(N_Q):
            facc = sendB_ref[pl.ds(q * N_Z + my_z, 1)].astype(
                jnp.float32).reshape(chunk, nh)
            for s in range(1, N_Z):
                zs = (my_z - s) % N_Z
                slot = q * N_Z + zs
                brecv = pltpu.make_async_remote_copy(
                    src_ref=sendB_ref.at[pl.ds(slot, 1)],
                    dst_ref=commB_ref.at[pl.ds(slot, 1)],
                    send_sem=bs_sems.at[slot],
                    recv_sem=br_sems.at[slot],
                    device_id=(0,),
                    device_id_type=pl.DeviceIdType.MESH,
                )
                brecv.wait_recv()
                facc = facc + commB_ref[pl.ds(slot, 1)].astype(
                    jnp.float32).reshape(chunk, nh)
            out_ref[:, q * nh:(q + 1) * nh] = facc

        for rdma in rdmas:
            rdma.wait_send()

    return pl.pallas_call(
        body,
        out_shape=jax.ShapeDtypeStruct((chunk, n), jnp.float32),
        in_specs=[
            pl.BlockSpec(memory_space=pltpu.VMEM),
            pl.BlockSpec(memory_space=pltpu.VMEM),
        ],
        out_specs=pl.BlockSpec(memory_space=pltpu.VMEM),
        scratch_shapes=[
            pltpu.VMEM((m, k), jnp.bfloat16),
            pltpu.VMEM((k, n), jnp.bfloat16),
            pltpu.VMEM((N_Q, N_Y, N_Z, chunk, nh), jnp.bfloat16),
            pltpu.VMEM((N_Q, N_Y, N_Z, chunk, nh), jnp.bfloat16),
            pltpu.VMEM((N_Q, N_Y, N_Z, chunk, nh), jnp.bfloat16),
            pltpu.VMEM((N_Q * N_Y, N_Z, chunk, nh), jnp.bfloat16),
            pltpu.VMEM((N_Q * N_Y, N_Z, chunk, nh), jnp.bfloat16),
            pltpu.VMEM((N_Q * N_Z, chunk, nh), jnp.bfloat16),
            pltpu.VMEM((N_Q * N_Z, chunk, nh), jnp.bfloat16),
            pltpu.SemaphoreType.DMA((N_Q,)),
            pltpu.SemaphoreType.DMA((N_Q,)),
            pltpu.SemaphoreType.DMA((N_Q * N_Y,)),
            pltpu.SemaphoreType.DMA((N_Q * N_Y,)),
            pltpu.SemaphoreType.DMA((N_Q * N_Z,)),
            pltpu.SemaphoreType.DMA((N_Q * N_Z,)),
        ],
        compiler_params=pltpu.CompilerParams(collective_id=0),
    )(A, B)


# device time: 30261 ns/iter; 1.2178x vs baseline; 1.0139x over previous
import jax
import jax.numpy as jnp
from jax import lax
from jax.experimental import pallas as pl
from jax.experimental.pallas import tpu as pltpu

N_DEV = 32
N_PLANE = 8
N_Z = 4
N_Y = 4
N_Q = 8


def kernel(A, B):
    m, k = A.shape
    _, n = B.shape
    chunk = m // N_DEV
    nh = n // N_Q
    half_rows = m // 2

    def body(a_ref, b_ref, out_ref, ab_ref, bb_ref,
             sendA1_ref, commA1_ref, ownA_ref,
             sendA2_ref, commA2_ref, sendB_ref, commB_ref,
             a1s_sems, a1r_sems, a2s_sems, a2r_sems, bs_sems, br_sems):
        my_id = lax.axis_index("i")
        my_z = my_id // N_PLANE
        my_p = my_id % N_PLANE
        my_y = my_p // 2
        my_x = (my_p & 1) ^ (my_y & 1)
        partner_p = my_p ^ 1

        def p_of(x, y):
            return 2 * y + (x ^ (y & 1))

        rail_peers = [my_z * N_PLANE + p_of(my_x, (my_y + s) % N_Y)
                      for s in range(1, N_Y)]
        z_peers = [((my_z + s) % N_Z) * N_PLANE + my_p
                   for s in range(1, N_Z)]
        peers = [my_z * N_PLANE + partner_p] + rail_peers + z_peers

        barrier_sem = pltpu.get_barrier_semaphore()
        for peer in peers:
            pl.semaphore_signal(
                barrier_sem, inc=1,
                device_id=(peer,), device_id_type=pl.DeviceIdType.MESH,
            )

        bb_ref[...] = b_ref[...].astype(jnp.bfloat16)
        for side in range(2):
            for y in range(N_Y):
                for zo in range(N_Z):
                    po = p_of((1 - my_x) if side == 0 else my_x, y)
                    dst = side * half_rows + (y * N_Z + zo) * chunk
                    ab_ref[dst:dst + chunk, :] = (
                        a_ref[pl.ds(zo * (m // N_Z) + po * chunk, chunk), :]
                        .astype(jnp.bfloat16)
                    )

        rdmas = []
        for q in range(N_Q):
            blk_opp = jnp.dot(
                ab_ref[:half_rows, :], bb_ref[:, q * nh:(q + 1) * nh],
                preferred_element_type=jnp.float32,
            )
            sendA1_ref[q] = blk_opp.astype(jnp.bfloat16).reshape(
                N_Y, N_Z, chunk, nh)
            if q == 0:
                pl.semaphore_wait(barrier_sem, len(peers))
            a1 = pltpu.make_async_remote_copy(
                src_ref=sendA1_ref.at[q],
                dst_ref=commA1_ref.at[q],
                send_sem=a1s_sems.at[q],
                recv_sem=a1r_sems.at[q],
                device_id=(my_z * N_PLANE + partner_p,),
                device_id_type=pl.DeviceIdType.MESH,
            )
            a1.start()
            rdmas.append(a1)
            blk_own = jnp.dot(
                ab_ref[half_rows:, :], bb_ref[:, q * nh:(q + 1) * nh],
                preferred_element_type=jnp.float32,
            )
            ownA_ref[q] = blk_own.astype(jnp.bfloat16).reshape(
                N_Y, N_Z, chunk, nh)

        for q in range(N_Q):
            a1recv = pltpu.make_async_remote_copy(
                src_ref=sendA1_ref.at[q],
                dst_ref=commA1_ref.at[q],
                send_sem=a1s_sems.at[q],
                recv_sem=a1r_sems.at[q],
                device_id=(0,),
                device_id_type=pl.DeviceIdType.MESH,
            )
            a1recv.wait_recv()
            pair = (commA1_ref[q].astype(jnp.float32)
                    + ownA_ref[q].astype(jnp.float32))
            sendA2_ref[q * N_Y:(q + 1) * N_Y] = pair.astype(jnp.bfloat16)
            for s in range(1, N_Y):
                yt = (my_y + s) % N_Y
                a2 = pltpu.make_async_remote_copy(
                    src_ref=sendA2_ref.at[pl.ds(q * N_Y + yt, 1)],
                    dst_ref=commA2_ref.at[pl.ds(q * N_Y + my_y, 1)],
                    send_sem=a2s_sems.at[q * N_Y + yt],
                    recv_sem=a2r_sems.at[q * N_Y + my_y],
                    device_id=(my_z * N_PLANE + p_of(my_x, yt),),
                    device_id_type=pl.DeviceIdType.MESH,
                )
                a2.start()
                rdmas.append(a2)

        for q in range(N_Q):
            acc = sendA2_ref[pl.ds(q * N_Y + my_y, 1)].astype(
                jnp.float32).reshape(N_Z, chunk, nh)
            for s in range(1, N_Y):
                ys = (my_y - s) % N_Y
                slot = q * N_Y + ys
                a2recv = pltpu.make_async_remote_copy(
                    src_ref=sendA2_ref.at[pl.ds(slot, 1)],
                    dst_ref=commA2_ref.at[pl.ds(slot, 1)],
                    send_sem=a2s_sems.at[slot],
                    recv_sem=a2r_sems.at[slot],
                    device_id=(0,),
                    device_id_type=pl.DeviceIdType.MESH,
                )
                a2recv.wait_recv()
                acc = acc + commA2_ref[pl.ds(slot, 1)].astype(
                    jnp.float32).reshape(N_Z, chunk, nh)
            sendB_ref[q * N_Z:(q + 1) * N_Z] = acc.astype(jnp.bfloat16)
            for s in range(1, N_Z):
                zt = (my_z + s) % N_Z
                b_rdma = pltpu.make_async_remote_copy(
                    src_ref=sendB_ref.at[pl.ds(q * N_Z + zt, 1)],
                    dst_ref=commB_ref.at[pl.ds(q * N_Z + my_z, 1)],
                    send_sem=bs_sems.at[q * N_Z + zt],
                    recv_sem=br_sems.at[q * N_Z + my_z],
                    device_id=(zt * N_PLANE + my_p,),
                    device_id_type=pl.DeviceIdType.MESH,
                )
                b_rdma.start()
                rdmas.append(b_rdma)

        for q in range(N_Q):
            facc = sendB_ref[pl.ds(q * N_Z + my_z, 1)].astype(
                jnp.float32).reshape(chunk, nh)
            for s in range(1, N_Z):
                zs = (my_z - s) % N_Z
                slot = q * N_Z + zs
                brecv = pltpu.make_async_remote_copy(
                    src_ref=sendB_ref.at[pl.ds(slot, 1)],
                    dst_ref=commB_ref.at[pl.ds(slot, 1)],
                    send_sem=bs_sems.at[slot],
                    recv_sem=br_sems.at[slot],
                    device_id=(0,),
                    device_id_type=pl.DeviceIdType.MESH,
                )
                brecv.wait_recv()
                facc = facc + commB_ref[pl.ds(slot, 1)].astype(
                    jnp.float32).reshape(chunk, nh)
            out_ref[:, q * nh:(q + 1) * nh] = facc

        for rdma in rdmas:
            rdma.wait_send()

    return pl.pallas_call(
        body,
        out_shape=jax.ShapeDtypeStruct((chunk, n), jnp.float32),
        in_specs=[
            pl.BlockSpec(memory_space=pltpu.VMEM),
            pl.BlockSpec(memory_space=pltpu.VMEM),
        ],
        out_specs=pl.BlockSpec(memory_space=pltpu.VMEM),
        scratch_shapes=[
            pltpu.VMEM((m, k), jnp.bfloat16),
            pltpu.VMEM((k, n), jnp.bfloat16),
            pltpu.VMEM((N_Q, N_Y, N_Z, chunk, nh), jnp.bfloat16),
            pltpu.VMEM((N_Q, N_Y, N_Z, chunk, nh), jnp.bfloat16),
            pltpu.VMEM((N_Q, N_Y, N_Z, chunk, nh), jnp.bfloat16),
            pltpu.VMEM((N_Q * N_Y, N_Z, chunk, nh), jnp.bfloat16),
            pltpu.VMEM((N_Q * N_Y, N_Z, chunk, nh), jnp.bfloat16),
            pltpu.VMEM((N_Q * N_Z, chunk, nh), jnp.bfloat16),
            pltpu.VMEM((N_Q * N_Z, chunk, nh), jnp.bfloat16),
            pltpu.SemaphoreType.DMA((N_Q,)),
            pltpu.SemaphoreType.DMA((N_Q,)),
            pltpu.SemaphoreType.DMA((N_Q * N_Y,)),
            pltpu.SemaphoreType.DMA((N_Q * N_Y,)),
            pltpu.SemaphoreType.DMA((N_Q * N_Z,)),
            pltpu.SemaphoreType.DMA((N_Q * N_Z,)),
        ],
        compiler_params=pltpu.CompilerParams(collective_id=0),
    )(A, B)
